# baseline (speedup 1.0000x reference)
import jax
import jax.numpy as jnp
from jax import lax
from jax.experimental import pallas as pl
from jax.experimental.pallas import tpu as pltpu
from jax.experimental.pallas import tpu_sc as plsc

NUM_CORES = 2
NUM_SUBCORES = 16
NW = NUM_CORES * NUM_SUBCORES

CHUNK = 512
D = 32
LANES = 16


def _transpose_chunk(buf, buf_t):
    base_rows = lax.iota(jnp.int32, LANES)
    cols = [jnp.full((LANES,), d, dtype=jnp.int32) for d in range(D)]

    @pl.loop(0, CHUNK // LANES)
    def _(g):
        rows = base_rows + g * LANES
        for d in range(D):
            v = plsc.load_gather(buf, [rows, cols[d]])
            buf_t[d, pl.ds(g * LANES, LANES)] = v


def _gather_body(idx_hbm, table_hbm, out_hbm, idx_v, buf0, buf1, bt0, bt1,
                 g0, g1, s0, s1):
    wid = lax.axis_index("s") * NUM_CORES + lax.axis_index("c")
    n_chunks = idx_hbm.shape[1]
    bufs, bts, gsems, ssems = (buf0, buf1), (bt0, bt1), (g0, g1), (s0, s1)
    cpb = 4096 // CHUNK

    pltpu.sync_copy(idx_hbm.at[wid], idx_v)

    def chunk_dst(j):
        gc = wid * n_chunks + j
        h = gc // cpb
        b0 = (gc % cpb) * CHUNK
        return out_hbm.at[h, :, pl.ds(b0, CHUNK)]

    def gather(j, k):
        pltpu.async_copy(table_hbm.at[idx_v.at[j]], bufs[k], gsems[k])

    def wait_gather(k):
        pltpu.make_async_copy(table_hbm.at[idx_v.at[0]], bufs[k], gsems[k]).wait()

    def store(j, k):
        pltpu.async_copy(bts[k], chunk_dst(j), ssems[k])

    def wait_store(j, k):
        pltpu.make_async_copy(bts[k], chunk_dst(j), ssems[k]).wait()

    for k in range(2):
        gather(k, k)
    for k in range(2):
        wait_gather(k)
        _transpose_chunk(bufs[k], bts[k])
        store(k, k)
        gather(2 + k, k)

    @pl.loop(2, n_chunks - 2, step=2)
    def _(j):
        for k in range(2):
            wait_gather(k)
            wait_store(j + k - 2, k)
            _transpose_chunk(bufs[k], bts[k])
            store(j + k, k)
            gather(j + k + 2, k)

    base = n_chunks - 2
    for k in range(2):
        wait_gather(k)
        wait_store(base + k - 2, k)
        _transpose_chunk(bufs[k], bts[k])
        store(base + k, k)
    for k in range(2):
        wait_store(base + k, k)


def kernel(embed_id, weight):
    batch, hist = embed_id.shape
    total = batch * hist
    assert total % (NW * CHUNK) == 0 and batch % CHUNK == 0
    n_chunks = total // (NW * CHUNK)

    idx3 = embed_id.T.astype(jnp.int32).reshape(NW, n_chunks, CHUNK)

    mesh = plsc.VectorSubcoreMesh(
        core_axis_name="c", subcore_axis_name="s",
        num_cores=NUM_CORES, num_subcores=NUM_SUBCORES,
    )
    run = pl.kernel(
        _gather_body,
        out_type=jax.ShapeDtypeStruct((hist, D, batch), jnp.float32),
        mesh=mesh,
        compiler_params=pltpu.CompilerParams(
            use_tc_tiling_on_sc=False, needs_layout_passes=False),
        scratch_types=(
            [pltpu.VMEM((n_chunks, CHUNK), jnp.int32)]
            + [pltpu.VMEM((CHUNK, D), jnp.float32) for _ in range(2)]
            + [pltpu.VMEM((D, CHUNK), jnp.float32) for _ in range(2)]
            + [pltpu.SemaphoreType.DMA for _ in range(4)]
        ),
    )
    out_t = run(idx3, weight)
    return out_t.transpose(2, 0, 1)


# speedup vs baseline: 1.2446x; 1.2446x over previous
import jax
import jax.numpy as jnp
from jax import lax
from jax.experimental import pallas as pl
from jax.experimental.pallas import tpu as pltpu
from jax.experimental.pallas import tpu_sc as plsc

NUM_CORES = 2
NUM_SUBCORES = 16
NW = NUM_CORES * NUM_SUBCORES

CHUNK = 1600
D = 32


def _gather_body(idx_hbm, table_hbm, out_hbm, idx_v, buf0, buf1, sem0, sem1):
    wid = lax.axis_index("s") * NUM_CORES + lax.axis_index("c")
    n_chunks = idx_hbm.shape[1]

    pltpu.sync_copy(idx_hbm.at[wid], idx_v)

    cp0 = pltpu.async_copy(table_hbm.at[idx_v.at[0]], buf0, sem0)
    cp1 = pltpu.async_copy(table_hbm.at[idx_v.at[1]], buf1, sem1)

    @pl.loop(0, n_chunks - 2, step=2)
    def _(base):
        pltpu.make_async_copy(table_hbm.at[idx_v.at[base]], buf0, sem0).wait()
        pltpu.sync_copy(buf0, out_hbm.at[wid, base])
        pltpu.async_copy(table_hbm.at[idx_v.at[base + 2]], buf0, sem0)
        pltpu.make_async_copy(table_hbm.at[idx_v.at[base + 1]], buf1, sem1).wait()
        pltpu.sync_copy(buf1, out_hbm.at[wid, base + 1])
        pltpu.async_copy(table_hbm.at[idx_v.at[base + 3]], buf1, sem1)

    cp0.wait()
    pltpu.sync_copy(buf0, out_hbm.at[wid, n_chunks - 2])
    cp1.wait()
    pltpu.sync_copy(buf1, out_hbm.at[wid, n_chunks - 1])


def kernel(embed_id, weight):
    batch, hist = embed_id.shape
    total = batch * hist
    assert total % (NW * CHUNK) == 0
    n_chunks = total // (NW * CHUNK)
    assert n_chunks % 2 == 0

    idx3 = embed_id.astype(jnp.int32).reshape(NW, n_chunks, CHUNK)

    mesh = plsc.VectorSubcoreMesh(
        core_axis_name="c", subcore_axis_name="s",
        num_cores=NUM_CORES, num_subcores=NUM_SUBCORES,
    )
    run = pl.kernel(
        _gather_body,
        out_type=jax.ShapeDtypeStruct((NW, n_chunks, CHUNK, D), jnp.float32),
        mesh=mesh,
        compiler_params=pltpu.CompilerParams(use_tc_tiling_on_sc=False),
        scratch_types=[
            pltpu.VMEM((n_chunks, CHUNK), jnp.int32),
            pltpu.VMEM((CHUNK, D), jnp.float32),
            pltpu.VMEM((CHUNK, D), jnp.float32),
            pltpu.SemaphoreType.DMA,
            pltpu.SemaphoreType.DMA,
        ],
    )
    out = run(idx3, weight)
    return out.reshape(batch, hist, D)


# speedup vs baseline: 1.3091x; 1.0519x over previous
import jax
import jax.numpy as jnp
from jax import lax
from jax.experimental import pallas as pl
from jax.experimental.pallas import tpu as pltpu
from jax.experimental.pallas import tpu_sc as plsc

NUM_CORES = 2
NUM_SUBCORES = 16
NW = NUM_CORES * NUM_SUBCORES

CHUNK = 1600
D = 32


def _gather_body(idx_hbm, table_hbm, out_hbm, idx_v, buf0, buf1, sem0, sem1):
    wid = lax.axis_index("s") * NUM_CORES + lax.axis_index("c")
    n_chunks = idx_hbm.shape[1]

    pltpu.sync_copy(idx_hbm.at[wid], idx_v)

    cp0 = pltpu.async_copy(table_hbm.at[idx_v.at[0]], buf0, sem0)
    cp1 = pltpu.async_copy(table_hbm.at[idx_v.at[1]], buf1, sem1)

    @pl.loop(0, n_chunks - 2, step=2)
    def _(base):
        pltpu.make_async_copy(table_hbm.at[idx_v.at[base]], buf0, sem0).wait()
        pltpu.sync_copy(buf0, out_hbm.at[wid, base])
        pltpu.async_copy(table_hbm.at[idx_v.at[base + 2]], buf0, sem0)
        pltpu.make_async_copy(table_hbm.at[idx_v.at[base + 1]], buf1, sem1).wait()
        pltpu.sync_copy(buf1, out_hbm.at[wid, base + 1])
        pltpu.async_copy(table_hbm.at[idx_v.at[base + 3]], buf1, sem1)

    cp0.wait()
    pltpu.sync_copy(buf0, out_hbm.at[wid, n_chunks - 2])
    cp1.wait()
    pltpu.sync_copy(buf1, out_hbm.at[wid, n_chunks - 1])


def kernel(embed_id, weight):
    batch, hist = embed_id.shape
    total = batch * hist
    assert total % (NW * CHUNK) == 0
    n_chunks = total // (NW * CHUNK)
    assert n_chunks % 2 == 0

    idx3 = embed_id.T.astype(jnp.int32).reshape(NW, n_chunks, CHUNK)

    mesh = plsc.VectorSubcoreMesh(
        core_axis_name="c", subcore_axis_name="s",
        num_cores=NUM_CORES, num_subcores=NUM_SUBCORES,
    )
    run = pl.kernel(
        _gather_body,
        out_type=jax.ShapeDtypeStruct((NW, n_chunks, CHUNK, D), jnp.float32),
        mesh=mesh,
        compiler_params=pltpu.CompilerParams(use_tc_tiling_on_sc=False),
        scratch_types=[
            pltpu.VMEM((n_chunks, CHUNK), jnp.int32),
            pltpu.VMEM((CHUNK, D), jnp.float32),
            pltpu.VMEM((CHUNK, D), jnp.float32),
            pltpu.SemaphoreType.DMA,
            pltpu.SemaphoreType.DMA,
        ],
    )
    out = run(idx3, weight)
    return out.reshape(hist, batch, D).transpose(1, 0, 2)
